# baseline (speedup 1.0000x reference)
import functools

import jax
import jax.numpy as jnp
from jax import lax
from jax.experimental import pallas as pl
from jax.experimental.pallas import tpu as pltpu
from jax.experimental.pallas import tpu_sc as plsc

_N = 10000
_E = 320000
_D = 128
_NCORE = 2
_NSUB = 16
_NW = _NCORE * _NSUB
_CH = 128
_NCH = 80
_NCHP = _NCH + 2
_EPW = _E // _NW
_NPAD = 10112
_RPT = _NPAD // _NSUB


def _seg_sum_sc(h, edges_r):
    mesh = plsc.VectorSubcoreMesh(core_axis_name="c", subcore_axis_name="s")

    @functools.partial(
        pl.kernel,
        mesh=mesh,
        out_type=jax.ShapeDtypeStruct((_NCORE, _NPAD, _D), jnp.float32),
        scratch_types=[
            pltpu.VMEM((_NCHP, _CH), jnp.int32),
            pltpu.VMEM((2, _CH), jnp.int32),
            pltpu.VMEM((2, _CH), jnp.int32),
            pltpu.VMEM((2, _CH, _D), jnp.float32),
            pltpu.VMEM_SHARED((_NPAD, _D), jnp.float32),
            pltpu.SemaphoreType.DMA((2,)),
        ],
    )
    def seg_kernel(h_hbm, edges_hbm, out_hbm, pk, sidx, didx, rowsb, acc,
                   sems):
        cid = lax.axis_index("c")
        sid = lax.axis_index("s")
        wid = sid * _NCORE + cid

        def zrow(i, carry):
            for j in range(_D // 16):
                rowsb[0, i, pl.ds(16 * j, 16)] = jnp.zeros((16,), jnp.float32)
            return carry

        lax.fori_loop(0, _CH, zrow, 0)
        base = sid * _RPT
        for k in range(_RPT // _CH):
            pltpu.sync_copy(rowsb.at[0], acc.at[pl.ds(base + k * _CH, _CH)])
        rem = _RPT % _CH
        if rem:
            pltpu.sync_copy(rowsb.at[0, pl.ds(0, rem)],
                            acc.at[pl.ds(base + (_RPT // _CH) * _CH, rem)])
        plsc.subcore_barrier()

        pltpu.sync_copy(edges_hbm.at[wid], pk)

        def unpack(j, par):
            for c in range(_CH // 16):
                v = pk[j, pl.ds(16 * c, 16)]
                sidx[par, pl.ds(16 * c, 16)] = v >> 14
                didx[par, pl.ds(16 * c, 16)] = v & 16383

        unpack(0, 0)
        pltpu.async_copy(h_hbm.at[sidx.at[0]], rowsb.at[0], sems.at[0])

        def body(j, carry):
            p = lax.rem(j, 2)
            pn = lax.rem(j + 1, 2)
            unpack(j + 1, pn)
            pltpu.async_copy(h_hbm.at[sidx.at[pn]], rowsb.at[pn],
                             sems.at[pn])
            pltpu.make_async_copy(h_hbm.at[sidx.at[p]], rowsb.at[p],
                                  sems.at[p]).wait()
            pltpu.sync_copy(rowsb.at[p], acc.at[didx.at[p]], add=True)
            return carry

        lax.fori_loop(0, _NCH, body, 0)
        pltpu.make_async_copy(h_hbm.at[sidx.at[0]], rowsb.at[_NCH % 2],
                              sems.at[_NCH % 2]).wait()
        plsc.subcore_barrier()

        pltpu.sync_copy(acc.at[pl.ds(base, _RPT)],
                        out_hbm.at[cid, pl.ds(base, _RPT)])

    return seg_kernel(h, edges_r)


_BM = 2000


def _full(shape):
    return pl.BlockSpec(shape, lambda i: (0, 0))


def _pre_tc(x, w, b):
    def body(x_ref, w_ref, b_ref, o_ref):
        o_ref[...] = (
            jnp.dot(x_ref[...], w_ref[...], preferred_element_type=jnp.float32)
            + b_ref[...]
        )

    return pl.pallas_call(
        body,
        grid=(_N // _BM,),
        in_specs=[
            pl.BlockSpec((_BM, _D), lambda i: (i, 0)),
            _full((_D, _D)),
            _full((1, _D)),
        ],
        out_specs=pl.BlockSpec((_BM, _D), lambda i: (i, 0)),
        out_shape=jax.ShapeDtypeStruct((_N, _D), jnp.float32),
    )(x, w, b.reshape(1, _D))


def _mlp_tc(h, agg, w1, b1, w2, b2):
    def body(h_ref, a0_ref, a1_ref, w1_ref, b1_ref, w2_ref, b2_ref, o_ref):
        z = h_ref[...] + a0_ref[...] + a1_ref[...]
        z = jnp.maximum(
            jnp.dot(z, w1_ref[...], preferred_element_type=jnp.float32)
            + b1_ref[...],
            0.0,
        )
        z = (
            jnp.dot(z, w2_ref[...], preferred_element_type=jnp.float32)
            + b2_ref[...]
        )
        o_ref[...] = jnp.maximum(z, 0.0)

    return pl.pallas_call(
        body,
        grid=(_N // _BM,),
        in_specs=[
            pl.BlockSpec((_BM, _D), lambda i: (i, 0)),
            pl.BlockSpec((_BM, _D), lambda i: (i, 0)),
            pl.BlockSpec((_BM, _D), lambda i: (i, 0)),
            _full((_D, _D)),
            _full((1, _D)),
            _full((_D, _D)),
            _full((1, _D)),
        ],
        out_specs=pl.BlockSpec((_BM, _D), lambda i: (i, 0)),
        out_shape=jax.ShapeDtypeStruct((_N, _D), jnp.float32),
    )(h, agg[0], agg[1], w1, b1.reshape(1, _D), w2, b2.reshape(1, _D))


def _post_tc(h, wp, bp, wr_pad, br_pad):
    def body(h_ref, wp_ref, bp_ref, wr_ref, br_ref, o_ref):
        t = jnp.maximum(
            jnp.dot(h_ref[...], wp_ref[...], preferred_element_type=jnp.float32)
            + bp_ref[...],
            0.0,
        )
        z = (
            jnp.dot(t, wr_ref[...], preferred_element_type=jnp.float32)
            + br_ref[...]
        )
        m = jnp.max(z, axis=1, keepdims=True)
        lse = jnp.log(jnp.sum(jnp.exp(z - m), axis=1, keepdims=True)) + m
        o_ref[...] = z - lse

    return pl.pallas_call(
        body,
        grid=(_N // _BM,),
        in_specs=[
            pl.BlockSpec((_BM, _D), lambda i: (i, 0)),
            _full((_D, _D)),
            _full((1, _D)),
            _full((_D, _D)),
            _full((1, _D)),
        ],
        out_specs=pl.BlockSpec((_BM, _D), lambda i: (i, 0)),
        out_shape=jax.ShapeDtypeStruct((_N, _D), jnp.float32),
    )(h, wp, bp.reshape(1, _D), wr_pad, br_pad)


def kernel(x, edge_index, W_pre, b_pre, W1s, b1s, W2s, b2s, W_post, b_post,
           W_ro, b_ro):
    src = edge_index[0]
    dst = edge_index[1]
    npad = _NCHP * _CH - _EPW
    packed = jnp.bitwise_or(jnp.left_shift(src, 14), dst)
    edges_r = jnp.concatenate(
        [packed.reshape(_NW, _EPW), jnp.full((_NW, npad), _N, jnp.int32)],
        axis=1).reshape(_NW, _NCHP, _CH)

    h = _pre_tc(x, W_pre, b_pre)
    for l in range(3):
        agg = _seg_sum_sc(h, edges_r)
        h = _mlp_tc(h, agg, W1s[l], b1s[l], W2s[l], b2s[l])

    nclass = W_ro.shape[1]
    wr_pad = jnp.zeros((_D, _D), jnp.float32).at[:, :nclass].set(W_ro)
    br_pad = jnp.full((1, _D), -1e30, jnp.float32).at[0, :nclass].set(b_ro)
    out = _post_tc(h, W_post, b_post, wr_pad, br_pad)[:, :nclass]
    return (out, h, h)


# speedup vs baseline: 1.0004x; 1.0004x over previous
import functools

import jax
import jax.numpy as jnp
from jax import lax
from jax.experimental import pallas as pl
from jax.experimental.pallas import tpu as pltpu
from jax.experimental.pallas import tpu_sc as plsc

_N = 10000
_E = 320000
_D = 128
_NCORE = 2
_NSUB = 16
_NW = _NCORE * _NSUB
_CH = 128
_NCH = 80
_NCHP = _NCH + 2
_EPW = _E // _NW
_NPAD = 10112
_RPT = _NPAD // _NSUB


def _seg_sum_sc(h, edges_r):
    mesh = plsc.VectorSubcoreMesh(core_axis_name="c", subcore_axis_name="s")

    @functools.partial(
        pl.kernel,
        mesh=mesh,
        out_type=jax.ShapeDtypeStruct((_NCORE, _NPAD, _D), jnp.float32),
        scratch_types=[
            pltpu.VMEM((_NCHP, _CH), jnp.int32),
            pltpu.VMEM((2, _CH), jnp.int32),
            pltpu.VMEM((2, _CH), jnp.int32),
            pltpu.VMEM((2, _CH, _D), jnp.float32),
            pltpu.VMEM_SHARED((_NPAD, _D), jnp.float32),
            pltpu.SemaphoreType.DMA((2,)),
            pltpu.SemaphoreType.DMA((2,)),
        ],
    )
    def seg_kernel(h_hbm, edges_hbm, out_hbm, pk, sidx, didx, rowsb, acc,
                   gsems, ssems):
        cid = lax.axis_index("c")
        sid = lax.axis_index("s")
        wid = sid * _NCORE + cid

        def zrow(i, carry):
            for j in range(_D // 16):
                rowsb[0, i, pl.ds(16 * j, 16)] = jnp.zeros((16,), jnp.float32)
            return carry

        lax.fori_loop(0, _CH, zrow, 0)
        base = sid * _RPT
        for k in range(_RPT // _CH):
            pltpu.sync_copy(rowsb.at[0], acc.at[pl.ds(base + k * _CH, _CH)])
        rem = _RPT % _CH
        if rem:
            pltpu.sync_copy(rowsb.at[0, pl.ds(0, rem)],
                            acc.at[pl.ds(base + (_RPT // _CH) * _CH, rem)])
        plsc.subcore_barrier()

        pltpu.sync_copy(edges_hbm.at[wid], pk)

        def unpack(j, par):
            for c in range(_CH // 16):
                v = pk[j, pl.ds(16 * c, 16)]
                sidx[par, pl.ds(16 * c, 16)] = v >> 14
                didx[par, pl.ds(16 * c, 16)] = v & 16383

        unpack(0, 0)
        pltpu.async_copy(h_hbm.at[sidx.at[0]], rowsb.at[0], gsems.at[0])
        unpack(1, 1)
        pltpu.async_copy(h_hbm.at[sidx.at[1]], rowsb.at[1], gsems.at[1])
        pltpu.make_async_copy(h_hbm.at[sidx.at[0]], rowsb.at[0],
                              gsems.at[0]).wait()
        pltpu.async_copy(rowsb.at[0], acc.at[didx.at[0]], ssems.at[0],
                         add=True)

        def body(j, carry):
            p = lax.rem(j, 2)
            pn = lax.rem(j + 1, 2)
            pltpu.make_async_copy(rowsb.at[pn], acc.at[didx.at[pn]],
                                  ssems.at[pn]).wait()
            unpack(j + 1, pn)
            pltpu.async_copy(h_hbm.at[sidx.at[pn]], rowsb.at[pn],
                             gsems.at[pn])
            pltpu.make_async_copy(h_hbm.at[sidx.at[p]], rowsb.at[p],
                                  gsems.at[p]).wait()
            pltpu.async_copy(rowsb.at[p], acc.at[didx.at[p]], ssems.at[p],
                             add=True)
            return carry

        lax.fori_loop(1, _NCH, body, 0)
        pltpu.make_async_copy(h_hbm.at[sidx.at[0]], rowsb.at[_NCH % 2],
                              gsems.at[_NCH % 2]).wait()
        pltpu.make_async_copy(rowsb.at[(_NCH - 1) % 2],
                              acc.at[didx.at[(_NCH - 1) % 2]],
                              ssems.at[(_NCH - 1) % 2]).wait()
        plsc.subcore_barrier()

        pltpu.sync_copy(acc.at[pl.ds(base, _RPT)],
                        out_hbm.at[cid, pl.ds(base, _RPT)])

    return seg_kernel(h, edges_r)


_BM = 2000


def _full(shape):
    return pl.BlockSpec(shape, lambda i: (0, 0))


def _pre_tc(x, w, b):
    def body(x_ref, w_ref, b_ref, o_ref):
        o_ref[...] = (
            jnp.dot(x_ref[...], w_ref[...], preferred_element_type=jnp.float32)
            + b_ref[...]
        )

    return pl.pallas_call(
        body,
        grid=(_N // _BM,),
        in_specs=[
            pl.BlockSpec((_BM, _D), lambda i: (i, 0)),
            _full((_D, _D)),
            _full((1, _D)),
        ],
        out_specs=pl.BlockSpec((_BM, _D), lambda i: (i, 0)),
        out_shape=jax.ShapeDtypeStruct((_N, _D), jnp.float32),
    )(x, w, b.reshape(1, _D))


def _mlp_tc(h, agg, w1, b1, w2, b2):
    def body(h_ref, a0_ref, a1_ref, w1_ref, b1_ref, w2_ref, b2_ref, o_ref):
        z = h_ref[...] + a0_ref[...] + a1_ref[...]
        z = jnp.maximum(
            jnp.dot(z, w1_ref[...], preferred_element_type=jnp.float32)
            + b1_ref[...],
            0.0,
        )
        z = (
            jnp.dot(z, w2_ref[...], preferred_element_type=jnp.float32)
            + b2_ref[...]
        )
        o_ref[...] = jnp.maximum(z, 0.0)

    return pl.pallas_call(
        body,
        grid=(_N // _BM,),
        in_specs=[
            pl.BlockSpec((_BM, _D), lambda i: (i, 0)),
            pl.BlockSpec((_BM, _D), lambda i: (i, 0)),
            pl.BlockSpec((_BM, _D), lambda i: (i, 0)),
            _full((_D, _D)),
            _full((1, _D)),
            _full((_D, _D)),
            _full((1, _D)),
        ],
        out_specs=pl.BlockSpec((_BM, _D), lambda i: (i, 0)),
        out_shape=jax.ShapeDtypeStruct((_N, _D), jnp.float32),
    )(h, agg[0], agg[1], w1, b1.reshape(1, _D), w2, b2.reshape(1, _D))


def _post_tc(h, wp, bp, wr_pad, br_pad):
    def body(h_ref, wp_ref, bp_ref, wr_ref, br_ref, o_ref):
        t = jnp.maximum(
            jnp.dot(h_ref[...], wp_ref[...], preferred_element_type=jnp.float32)
            + bp_ref[...],
            0.0,
        )
        z = (
            jnp.dot(t, wr_ref[...], preferred_element_type=jnp.float32)
            + br_ref[...]
        )
        m = jnp.max(z, axis=1, keepdims=True)
        lse = jnp.log(jnp.sum(jnp.exp(z - m), axis=1, keepdims=True)) + m
        o_ref[...] = z - lse

    return pl.pallas_call(
        body,
        grid=(_N // _BM,),
        in_specs=[
            pl.BlockSpec((_BM, _D), lambda i: (i, 0)),
            _full((_D, _D)),
            _full((1, _D)),
            _full((_D, _D)),
            _full((1, _D)),
        ],
        out_specs=pl.BlockSpec((_BM, _D), lambda i: (i, 0)),
        out_shape=jax.ShapeDtypeStruct((_N, _D), jnp.float32),
    )(h, wp, bp.reshape(1, _D), wr_pad, br_pad)


def kernel(x, edge_index, W_pre, b_pre, W1s, b1s, W2s, b2s, W_post, b_post,
           W_ro, b_ro):
    src = edge_index[0]
    dst = edge_index[1]
    npad = _NCHP * _CH - _EPW
    packed = jnp.bitwise_or(jnp.left_shift(src, 14), dst)
    edges_r = jnp.concatenate(
        [packed.reshape(_NW, _EPW), jnp.full((_NW, npad), _N, jnp.int32)],
        axis=1).reshape(_NW, _NCHP, _CH)

    h = _pre_tc(x, W_pre, b_pre)
    for l in range(3):
        agg = _seg_sum_sc(h, edges_r)
        h = _mlp_tc(h, agg, W1s[l], b1s[l], W2s[l], b2s[l])

    nclass = W_ro.shape[1]
    wr_pad = jnp.zeros((_D, _D), jnp.float32).at[:, :nclass].set(W_ro)
    br_pad = jnp.full((1, _D), -1e30, jnp.float32).at[0, :nclass].set(b_ro)
    out = _post_tc(h, W_post, b_post, wr_pad, br_pad)[:, :nclass]
    return (out, h, h)
